# baseline (speedup 1.0000x reference)
import jax
import jax.numpy as jnp
from jax.experimental import pallas as pl
from jax.experimental.pallas import tpu as pltpu

_TM = 1024


def kernel(latents, actions, w1, b1, w2, b2, w3, b3):
    out_dtype = latents.dtype
    B, S, d_lat = latents.shape
    M = B * S
    hid = w1.shape[1]
    obs = w3.shape[1]
    num_actions = w1.shape[0] - d_lat

    if actions.ndim == 2:
        oh = jax.nn.one_hot(actions, num_actions, dtype=jnp.bfloat16)
    else:
        oh = actions.astype(jnp.bfloat16)
    d_act = oh.shape[-1]

    w1l = w1[:d_lat].astype(jnp.bfloat16)
    w1a = w1[d_lat:d_lat + d_act].astype(jnp.bfloat16)
    w2c = w2.astype(jnp.bfloat16)
    w3c = w3.astype(jnp.bfloat16)
    b1r = b1.astype(jnp.float32).reshape(1, hid)
    b2r = b2.astype(jnp.float32).reshape(1, hid)
    b3r = b3.astype(jnp.float32).reshape(1, obs)

    lat2 = latents.reshape(M, d_lat)
    oh2 = oh.reshape(M, d_act)

    tm = min(_TM, M)
    n_steps = M // tm
    mesh = pltpu.create_tensorcore_mesh("core", num_cores=2)

    def stateful(refs):
        (lat_ref, oh_ref, w1l_ref, w1a_ref, b1_ref, w2_ref, b2_ref,
         w3_ref, b3_ref, o_ref) = refs

        @pl.core_map(mesh)
        def _per_core():
            def scoped(w1l_v, w1a_v, b1_v, w2_v, b2_v, w3_v, b3_v, sem):
                copies = [
                    pltpu.make_async_copy(w1l_ref, w1l_v, sem),
                    pltpu.make_async_copy(w1a_ref, w1a_v, sem),
                    pltpu.make_async_copy(b1_ref, b1_v, sem),
                    pltpu.make_async_copy(w2_ref, w2_v, sem),
                    pltpu.make_async_copy(b2_ref, b2_v, sem),
                    pltpu.make_async_copy(w3_ref, w3_v, sem),
                    pltpu.make_async_copy(b3_ref, b3_v, sem),
                ]
                for c in copies:
                    c.start()
                for c in copies:
                    c.wait()

                def body(lat_v, oh_v, o_v):
                    lat = lat_v[...].astype(jnp.bfloat16)
                    h1 = jnp.dot(lat, w1l_v[...],
                                 preferred_element_type=jnp.float32)
                    h1 = h1 + jnp.dot(oh_v[...], w1a_v[...],
                                      preferred_element_type=jnp.float32)
                    h1 = jnp.maximum(h1 + b1_v[...], 0.0)
                    h2 = jnp.dot(h1.astype(jnp.bfloat16), w2_v[...],
                                 preferred_element_type=jnp.float32)
                    h2 = jnp.maximum(h2 + b2_v[...], 0.0)
                    out = jnp.dot(h2.astype(jnp.bfloat16), w3_v[...],
                                  preferred_element_type=jnp.float32)
                    o_v[...] = (out + b3_v[...]).astype(o_v.dtype)

                pipeline = pltpu.emit_pipeline(
                    body,
                    grid=(n_steps,),
                    in_specs=[
                        pl.BlockSpec((tm, d_lat), lambda i: (i, 0)),
                        pl.BlockSpec((tm, d_act), lambda i: (i, 0)),
                    ],
                    out_specs=[pl.BlockSpec((tm, obs), lambda i: (i, 0))],
                    core_axis_name="core",
                    dimension_semantics=(pltpu.PARALLEL,),
                )
                pipeline(lat_ref, oh_ref, o_ref)

            pl.run_scoped(
                scoped,
                pltpu.VMEM((d_lat, hid), jnp.bfloat16),
                pltpu.VMEM((d_act, hid), jnp.bfloat16),
                pltpu.VMEM((1, hid), jnp.float32),
                pltpu.VMEM((hid, hid), jnp.bfloat16),
                pltpu.VMEM((1, hid), jnp.float32),
                pltpu.VMEM((hid, obs), jnp.bfloat16),
                pltpu.VMEM((1, obs), jnp.float32),
                pltpu.SemaphoreType.DMA,
            )

    init = (lat2, oh2, w1l, w1a, b1r, w2c, b2r, w3c, b3r,
            jax.lax.empty((M, obs), out_dtype))
    final = pl.run_state(stateful)(init)
    return final[-1].reshape(B, S, obs)
